# baseline (speedup 1.0000x reference)
import functools

import jax
import jax.numpy as jnp
from jax import lax
from jax.experimental import pallas as pl
from jax.experimental.pallas import tpu as pltpu
from jax.experimental.pallas import tpu_sc as plsc

_D = 64
_NW = 32
_GRP = 4
_SPLITS = ((0, 128), (128, 72))


@functools.lru_cache(maxsize=None)
def _make_gather(nb: int, hist: int):
    rpw = nb // _NW
    rows_per_grp = _GRP // len(_SPLITS)
    ngrp = rpw // rows_per_grp
    assert ngrp >= 3
    mesh = plsc.VectorSubcoreMesh(core_axis_name="c", subcore_axis_name="s")

    @functools.partial(
        pl.kernel,
        mesh=mesh,
        compiler_params=pltpu.CompilerParams(use_tc_tiling_on_sc=False),
        out_type=jax.ShapeDtypeStruct((nb, hist, 2 * _D), jnp.float32),
        scratch_types=[
            pltpu.VMEM((rpw, hist), jnp.int32),
            pltpu.VMEM((2 * _GRP, max(w for _, w in _SPLITS), _D), jnp.float32),
            pltpu.SemaphoreType.DMA((2 * _GRP,)),
            pltpu.SemaphoreType.DMA((2 * _GRP,)),
        ],
    )
    def k(table_hbm, idx_hbm, out_hbm, idx_v, buf, gs, ws):
        c = lax.axis_index("c")
        s = lax.axis_index("s")
        wid = s * 2 + c
        row0 = wid * rpw
        pltpu.sync_copy(idx_hbm.at[pl.ds(row0, rpw)], idx_v)

        def chunk(g, b):
            r = g * rows_per_grp + b // len(_SPLITS)
            h0, w = _SPLITS[b % len(_SPLITS)]
            return r, h0, w

        def gather(g, b, bb):
            r, h0, w = chunk(g, b)
            pltpu.async_copy(
                table_hbm.at[idx_v.at[r, pl.ds(h0, w)]],
                buf.at[bb, pl.ds(0, w)], gs.at[bb])

        def gwait(b, bb):
            _, _, w = chunk(0, b)
            pltpu.make_async_copy(
                table_hbm.at[idx_v.at[0, pl.ds(0, w)]],
                buf.at[bb, pl.ds(0, w)], gs.at[bb]).wait()

        def write(g, b, bb):
            r, h0, w = chunk(g, b)
            pltpu.async_copy(
                buf.at[bb, pl.ds(0, w)],
                out_hbm.at[row0 + r, pl.ds(h0, w), pl.ds(0, _D)], ws.at[bb])

        def wwait(b, bb):
            _, h0, w = chunk(0, b)
            pltpu.make_async_copy(
                buf.at[bb, pl.ds(0, w)],
                out_hbm.at[0, pl.ds(h0, w), pl.ds(0, _D)], ws.at[bb]).wait()

        for b in range(_GRP):
            gather(0, b, b)
        for b in range(_GRP):
            gather(1, b, _GRP + b)

        def body(g, carry):
            bs = (g % 2) * _GRP
            for b in range(_GRP):
                gwait(b, bs + b)
                write(g, b, bs + b)
            for b in range(_GRP):
                wwait(b, bs + b)
                gather(g + 2, b, bs + b)
            return carry

        lax.fori_loop(0, ngrp - 2, body, 0)

        for g in (ngrp - 2, ngrp - 1):
            bs = (g % 2) * _GRP
            for b in range(_GRP):
                gwait(b, bs + b)
                write(g, b, bs + b)
            for b in range(_GRP):
                wwait(b, bs + b)

    return k


_BB = 32


@functools.lru_cache(maxsize=None)
def _make_slice(nb: int, hist: int):
    def body(src_ref, dst_ref):
        dst_ref[...] = src_ref[:, :, : _D]

    return pl.pallas_call(
        body,
        grid=(nb // _BB,),
        in_specs=[pl.BlockSpec((_BB, hist, 2 * _D), lambda i: (i, 0, 0))],
        out_specs=pl.BlockSpec((_BB, hist, _D), lambda i: (i, 0, 0)),
        out_shape=jax.ShapeDtypeStruct((nb, hist, _D), jnp.float32),
    )


def kernel(indices, weight):
    nb, hist = indices.shape
    out = _make_gather(nb, hist)(weight, indices.astype(jnp.int32))
    return _make_slice(nb, hist)(out)


# speedup vs baseline: 1.1866x; 1.1866x over previous
import functools

import jax
import jax.numpy as jnp
from jax import lax
from jax.experimental import pallas as pl
from jax.experimental.pallas import tpu as pltpu
from jax.experimental.pallas import tpu_sc as plsc

_D = 64
_NW = 32
_GRP = 4
_SPLITS = ((0, 128), (128, 72))


@functools.lru_cache(maxsize=None)
def _make_gather(nb: int, hist: int):
    rpw = nb // _NW
    rows_per_grp = _GRP // len(_SPLITS)
    ngrp = rpw // rows_per_grp
    assert ngrp >= 3
    mesh = plsc.VectorSubcoreMesh(core_axis_name="c", subcore_axis_name="s")

    @functools.partial(
        pl.kernel,
        mesh=mesh,
        compiler_params=pltpu.CompilerParams(use_tc_tiling_on_sc=False),
        out_type=jax.ShapeDtypeStruct((nb, hist, 2 * _D), jnp.float32),
        scratch_types=[
            pltpu.VMEM((rpw, hist), jnp.int32),
            pltpu.VMEM((2 * _GRP, max(w for _, w in _SPLITS), _D), jnp.float32),
            pltpu.SemaphoreType.DMA((2 * _GRP,)),
            pltpu.SemaphoreType.DMA((2 * _GRP,)),
        ],
    )
    def k(table_hbm, idx_hbm, out_hbm, idx_v, buf, gs, ws):
        c = lax.axis_index("c")
        s = lax.axis_index("s")
        wid = s * 2 + c
        row0 = wid * rpw
        pltpu.sync_copy(idx_hbm.at[pl.ds(row0, rpw)], idx_v)

        def chunk(g, b):
            r = g * rows_per_grp + b // len(_SPLITS)
            h0, w = _SPLITS[b % len(_SPLITS)]
            return r, h0, w

        def gather(g, b, bb):
            r, h0, w = chunk(g, b)
            pltpu.async_copy(
                table_hbm.at[idx_v.at[r, pl.ds(h0, w)]],
                buf.at[bb, pl.ds(0, w)], gs.at[bb])

        def gwait(b, bb):
            _, _, w = chunk(0, b)
            pltpu.make_async_copy(
                table_hbm.at[idx_v.at[0, pl.ds(0, w)]],
                buf.at[bb, pl.ds(0, w)], gs.at[bb]).wait()

        def write(g, b, bb):
            r, h0, w = chunk(g, b)
            pltpu.async_copy(
                buf.at[bb, pl.ds(0, w)],
                out_hbm.at[row0 + r, pl.ds(h0, w), pl.ds(0, _D)], ws.at[bb])

        def wwait(b, bb):
            _, h0, w = chunk(0, b)
            pltpu.make_async_copy(
                buf.at[bb, pl.ds(0, w)],
                out_hbm.at[0, pl.ds(h0, w), pl.ds(0, _D)], ws.at[bb]).wait()

        for b in range(_GRP):
            gather(0, b, b)
        for b in range(_GRP):
            gather(1, b, _GRP + b)

        def body(g, carry):
            bs = (g % 2) * _GRP
            for b in range(_GRP):
                gwait(b, bs + b)
                write(g, b, bs + b)
            for b in range(_GRP):
                wwait(b, bs + b)
                gather(g + 2, b, bs + b)
            return carry

        lax.fori_loop(0, ngrp - 2, body, 0)

        for g in (ngrp - 2, ngrp - 1):
            bs = (g % 2) * _GRP
            for b in range(_GRP):
                gwait(b, bs + b)
                write(g, b, bs + b)
            for b in range(_GRP):
                wwait(b, bs + b)

    return k


def kernel(indices, weight):
    nb, hist = indices.shape
    out = _make_gather(nb, hist)(weight, indices.astype(jnp.int32))
    sel = jnp.eye(2 * _D, _D, dtype=jnp.float32)
    return jax.lax.dot_general(
        out, sel, (((2,), (0,)), ((), ())),
        precision=jax.lax.Precision.HIGHEST,
        preferred_element_type=jnp.float32)


# speedup vs baseline: 1.3950x; 1.1756x over previous
import functools

import jax
import jax.numpy as jnp
from jax import lax
from jax.experimental import pallas as pl
from jax.experimental.pallas import tpu as pltpu
from jax.experimental.pallas import tpu_sc as plsc

_D = 64
_NW = 32
_GRP = 4
_CW = 128


@functools.lru_cache(maxsize=None)
def _make_gather(nb: int, hist: int):
    npw = nb * hist // _NW
    nch = npw // _CW
    ngrp = nch // _GRP
    assert ngrp >= 3 and nch % _GRP == 0
    mesh = plsc.VectorSubcoreMesh(core_axis_name="c", subcore_axis_name="s")

    @functools.partial(
        pl.kernel,
        mesh=mesh,
        compiler_params=pltpu.CompilerParams(use_tc_tiling_on_sc=False),
        out_type=jax.ShapeDtypeStruct((nb * hist, 2 * _D), jnp.float32),
        scratch_types=[
            pltpu.VMEM((npw,), jnp.int32),
            pltpu.VMEM((2 * _GRP, _CW, _D), jnp.float32),
            pltpu.SemaphoreType.DMA((2 * _GRP,)),
            pltpu.SemaphoreType.DMA((2 * _GRP,)),
        ],
    )
    def k(table_hbm, idx_hbm, out_hbm, idx_v, buf, gs, ws):
        c = lax.axis_index("c")
        s = lax.axis_index("s")
        wid = s * 2 + c
        pos0 = wid * npw
        pltpu.sync_copy(idx_hbm.at[pl.ds(pos0, npw)], idx_v)

        def gather(ch, bb):
            pltpu.async_copy(
                table_hbm.at[idx_v.at[pl.ds(ch * _CW, _CW)]],
                buf.at[bb], gs.at[bb])

        def gwait(bb):
            pltpu.make_async_copy(
                table_hbm.at[idx_v.at[pl.ds(0, _CW)]],
                buf.at[bb], gs.at[bb]).wait()

        def write(ch, bb):
            pltpu.async_copy(
                buf.at[bb],
                out_hbm.at[pl.ds(pos0 + ch * _CW, _CW), pl.ds(0, _D)],
                ws.at[bb])

        def wwait(bb):
            pltpu.make_async_copy(
                buf.at[bb],
                out_hbm.at[pl.ds(0, _CW), pl.ds(0, _D)], ws.at[bb]).wait()

        for b in range(_GRP):
            gather(b, b)
        for b in range(_GRP):
            gather(_GRP + b, _GRP + b)

        def body(g, carry):
            bs = (g % 2) * _GRP
            for b in range(_GRP):
                gwait(bs + b)
                write(g * _GRP + b, bs + b)
            for b in range(_GRP):
                wwait(bs + b)
                gather((g + 2) * _GRP + b, bs + b)
            return carry

        lax.fori_loop(0, ngrp - 2, body, 0)

        for g in (ngrp - 2, ngrp - 1):
            bs = (g % 2) * _GRP
            for b in range(_GRP):
                gwait(bs + b)
                write(g * _GRP + b, bs + b)
            for b in range(_GRP):
                wwait(bs + b)

    return k


def kernel(indices, weight):
    nb, hist = indices.shape
    out = _make_gather(nb, hist)(weight, indices.reshape(-1).astype(jnp.int32))
    return out.reshape(nb, hist, 2 * _D)[:, :, :_D]


# speedup vs baseline: 1.3981x; 1.0022x over previous
import functools

import jax
import jax.numpy as jnp
from jax import lax
from jax.experimental import pallas as pl
from jax.experimental.pallas import tpu as pltpu
from jax.experimental.pallas import tpu_sc as plsc

_D = 64
_NW = 32
_GRP = 2
_CW = 256


@functools.lru_cache(maxsize=None)
def _make_gather(nb: int, hist: int):
    npw = nb * hist // _NW
    nch = npw // _CW
    ngrp = nch // _GRP
    assert ngrp >= 3 and nch % _GRP == 0
    mesh = plsc.VectorSubcoreMesh(core_axis_name="c", subcore_axis_name="s")

    @functools.partial(
        pl.kernel,
        mesh=mesh,
        compiler_params=pltpu.CompilerParams(use_tc_tiling_on_sc=False),
        out_type=jax.ShapeDtypeStruct((nb * hist, 2 * _D), jnp.float32),
        scratch_types=[
            pltpu.VMEM((npw,), jnp.int32),
            pltpu.VMEM((2 * _GRP, _CW, _D), jnp.float32),
            pltpu.SemaphoreType.DMA((2 * _GRP,)),
            pltpu.SemaphoreType.DMA((2 * _GRP,)),
        ],
    )
    def k(table_hbm, idx_hbm, out_hbm, idx_v, buf, gs, ws):
        c = lax.axis_index("c")
        s = lax.axis_index("s")
        wid = s * 2 + c
        pos0 = wid * npw
        pltpu.sync_copy(idx_hbm.at[pl.ds(pos0, npw)], idx_v)

        def gather(ch, bb):
            pltpu.async_copy(
                table_hbm.at[idx_v.at[pl.ds(ch * _CW, _CW)]],
                buf.at[bb], gs.at[bb])

        def gwait(bb):
            pltpu.make_async_copy(
                table_hbm.at[idx_v.at[pl.ds(0, _CW)]],
                buf.at[bb], gs.at[bb]).wait()

        def write(ch, bb):
            pltpu.async_copy(
                buf.at[bb],
                out_hbm.at[pl.ds(pos0 + ch * _CW, _CW), pl.ds(0, _D)],
                ws.at[bb])

        def wwait(bb):
            pltpu.make_async_copy(
                buf.at[bb],
                out_hbm.at[pl.ds(0, _CW), pl.ds(0, _D)], ws.at[bb]).wait()

        for b in range(_GRP):
            gather(b, b)
        for b in range(_GRP):
            gather(_GRP + b, _GRP + b)

        def body(g, carry):
            bs = (g % 2) * _GRP
            for b in range(_GRP):
                gwait(bs + b)
                write(g * _GRP + b, bs + b)
            for b in range(_GRP):
                wwait(bs + b)
                gather((g + 2) * _GRP + b, bs + b)
            return carry

        lax.fori_loop(0, ngrp - 2, body, 0)

        for g in (ngrp - 2, ngrp - 1):
            bs = (g % 2) * _GRP
            for b in range(_GRP):
                gwait(bs + b)
                write(g * _GRP + b, bs + b)
            for b in range(_GRP):
                wwait(bs + b)

    return k


def kernel(indices, weight):
    nb, hist = indices.shape
    out = _make_gather(nb, hist)(weight, indices.reshape(-1).astype(jnp.int32))
    return out.reshape(nb, hist, 2 * _D)[:, :, :_D]
